# Initial kernel scaffold; baseline (speedup 1.0000x reference)
import jax
import jax.numpy as jnp
from jax.experimental import pallas as pl


def kernel(queries, keys, values):
    raise NotImplementedError("write your pallas kernel here")


# speedup vs baseline: 1.6857x; 1.6857x over previous
import functools

import jax
import jax.numpy as jnp
from jax import lax
from jax.experimental import pallas as pl
from jax.experimental.pallas import tpu as pltpu
from jax.experimental.pallas import tpu_sc as plsc

_Q = 1024
_D = 16
_KB = 512
_BLUR = 0.01
_PAD = 1e18
_NC = 2
_NS = 16


def _tc_body(q_ref, kt_ref, keep_ref, idx_ref, racc, iacc, *, nsteps, kb):
    j = pl.program_id(0)
    kt = kt_ref[...]
    qm2 = q_ref[...] * (-2.0)
    m = jnp.dot(qm2, kt, preferred_element_type=jnp.float32)
    ksq = jnp.sum(kt * kt, axis=0, keepdims=True)
    s = m + ksq

    @pl.when(j == 0)
    def _():
        racc[...] = s
        iacc[...] = jnp.zeros_like(iacc)

    @pl.when(j > 0)
    def _():
        r = racc[...]
        better = s < r
        racc[...] = jnp.where(better, s, r)
        iacc[...] = jnp.where(better, j, iacc[...])

    @pl.when(j == nsteps - 1)
    def _():
        r = racc[...]
        smin = jnp.min(r, axis=1, keepdims=True)
        lane = lax.broadcasted_iota(jnp.int32, r.shape, 1)
        gidx = iacc[...] * kb + lane
        cand = jnp.where(r == smin, gidx, jnp.int32(2**31 - 1))
        idx_ref[...] = jnp.min(cand, axis=1, keepdims=True)
        q = q_ref[...]
        qsq = jnp.sum(q * q, axis=1, keepdims=True)
        loss = jnp.sqrt(jnp.maximum(qsq + smin, 1e-12))
        keep_ref[...] = jnp.where(loss <= _BLUR, 1.0, 0.0).astype(jnp.float32)


def _tc_min_argmin(queries, keys_t_padded, nsteps):
    keep, idx = pl.pallas_call(
        functools.partial(_tc_body, nsteps=nsteps, kb=_KB),
        grid=(nsteps,),
        in_specs=[
            pl.BlockSpec((_Q, _D), lambda j: (0, 0)),
            pl.BlockSpec((_D, _KB), lambda j: (0, j)),
        ],
        out_specs=[
            pl.BlockSpec((_Q, 1), lambda j: (0, 0)),
            pl.BlockSpec((_Q, 1), lambda j: (0, 0)),
        ],
        out_shape=[
            jax.ShapeDtypeStruct((_Q, 1), jnp.float32),
            jax.ShapeDtypeStruct((_Q, 1), jnp.int32),
        ],
        scratch_shapes=[
            pltpu.VMEM((_Q, _KB), jnp.float32),
            pltpu.VMEM((_Q, _KB), jnp.int32),
        ],
    )(queries, keys_t_padded)
    return keep, idx


_CH = _Q // (_NC * _NS)


@functools.cache
def _make_sc_gather_select():
    @functools.partial(
        pl.kernel,
        out_type=jax.ShapeDtypeStruct((_Q,), jnp.float32),
        mesh=plsc.VectorSubcoreMesh(core_axis_name="c", subcore_axis_name="s",
                                    num_cores=_NC, num_subcores=_NS),
        scratch_types=[
            pltpu.VMEM((_CH,), jnp.int32),
            pltpu.VMEM((_CH,), jnp.float32),
            pltpu.VMEM((_CH,), jnp.float32),
            pltpu.VMEM((_CH,), jnp.float32),
            pltpu.SemaphoreType.DMA,
        ],
    )
    def _sc_gather_select(keep_hbm, idx_hbm, values_hbm, out_hbm,
                          idx_v, gat_v, keep_v, out_v, sem):
        wid = lax.axis_index("s") * _NC + lax.axis_index("c")
        base = wid * _CH
        pltpu.sync_copy(idx_hbm.at[pl.ds(base, _CH)], idx_v)
        pltpu.async_copy(values_hbm.at[idx_v], gat_v, sem).wait()
        pltpu.sync_copy(keep_hbm.at[pl.ds(base, _CH)], keep_v)
        for i in range(_CH // 16):
            sl = pl.ds(i * 16, 16)
            out_v[sl] = keep_v[sl] * gat_v[sl]
        pltpu.sync_copy(out_v, out_hbm.at[pl.ds(base, _CH)])

    return _sc_gather_select


def kernel(queries, keys, values):
    k = keys.shape[0]
    nsteps = -(-k // _KB)
    kp = nsteps * _KB
    keys_t = jnp.pad(keys, ((0, kp - k), (0, 0)), constant_values=_PAD).T
    keep, idx = _tc_min_argmin(queries, keys_t, nsteps)
    return _make_sc_gather_select()(keep.reshape(_Q), idx.reshape(_Q), values)
